# baseline (speedup 1.0000x reference)
import functools

import jax
import jax.numpy as jnp
from jax import lax
from jax.experimental import pallas as pl
from jax.experimental.pallas import tpu as pltpu
from jax.experimental.pallas import tpu_sc as plsc

_B = 16384
_K = 50
_D = 32
_KP = 64
_BKP = _B * _KP
_NTILES = 32
_PAIRS_PER_TILE = _BKP // _NTILES
_ROWS_PER_TILE = _B // _NTILES
_W = 128
_NW = _PAIRS_PER_TILE // _W
_NG = _W // 16


def _tc_mlp(player_state, W1, b1, W2, b2, temperature):
    def body(ps_ref, w1_ref, b1_ref, w2_ref, b2_ref, t_ref, o_ref):
        h = jnp.maximum(
            jnp.dot(ps_ref[...], w1_ref[...].T,
                    preferred_element_type=jnp.float32) + b1_ref[...],
            0.0,
        )
        pe = jnp.dot(h, w2_ref[...].T,
                     preferred_element_type=jnp.float32) + b2_ref[...]
        o_ref[...] = pe / t_ref[0]

    return pl.pallas_call(
        body,
        in_specs=[
            pl.BlockSpec((_B, 10), lambda: (0, 0)),
            pl.BlockSpec((32, 10), lambda: (0, 0)),
            pl.BlockSpec((1, 32), lambda: (0, 0)),
            pl.BlockSpec((_D, 32), lambda: (0, 0)),
            pl.BlockSpec((1, _D), lambda: (0, 0)),
            pl.BlockSpec(memory_space=pltpu.SMEM),
        ],
        out_specs=pl.BlockSpec((_B, _D), lambda: (0, 0)),
        out_shape=jax.ShapeDtypeStruct((_B, _D), jnp.float32),
    )(player_state, W1, b1.reshape(1, 32), W2, b2.reshape(1, _D),
      temperature.reshape(1))


def _sc_fused_score(emb_table, flat_ids, pe):
    mesh = plsc.VectorSubcoreMesh(core_axis_name="c", subcore_axis_name="s")

    @functools.partial(
        pl.kernel,
        out_type=jax.ShapeDtypeStruct((_BKP,), jnp.float32),
        mesh=mesh,
        compiler_params=pltpu.CompilerParams(
            use_tc_tiling_on_sc=False, needs_layout_passes=False),
        scratch_types=[
            pltpu.VMEM((_ROWS_PER_TILE * _D,), jnp.float32),
            pltpu.VMEM((_W, _D), jnp.float32),
            pltpu.VMEM((_W, _D), jnp.float32),
            pltpu.VMEM((_W,), jnp.int32),
            pltpu.VMEM((_W,), jnp.int32),
            pltpu.VMEM((_PAIRS_PER_TILE,), jnp.float32),
            pltpu.SemaphoreType.DMA,
            pltpu.SemaphoreType.DMA,
            pltpu.SemaphoreType.DMA,
            pltpu.SemaphoreType.DMA,
            pltpu.SemaphoreType.DMA,
        ],
    )
    def fused_kernel(tab_hbm, ids_hbm, pe_hbm, out_hbm,
                     pe_v, tab0, tab1, ids0, ids1, sc_v,
                     pe_sem, tsem0, tsem1, isem0, isem1):
        tabs = (tab0, tab1)
        idss = (ids0, ids1)
        tsems = (tsem0, tsem1)
        isems = (isem0, isem1)

        cid = lax.axis_index("c")
        sid = lax.axis_index("s")
        wid = sid * 2 + cid
        pair_base = wid * _PAIRS_PER_TILE
        row_base = wid * _ROWS_PER_TILE

        def ids_dma(w, buf):
            off = jnp.minimum(pair_base + w * _W, _BKP - _W)
            return pltpu.make_async_copy(
                ids_hbm.at[pl.ds(off, _W)], idss[buf], isems[buf])

        def gather_dma(buf):
            return pltpu.make_async_copy(
                tab_hbm.at[idss[buf]], tabs[buf], tsems[buf])

        pe_cp = pltpu.make_async_copy(
            pe_hbm.at[pl.ds(row_base * _D, _ROWS_PER_TILE * _D)],
            pe_v, pe_sem)
        pe_cp.start()
        ids_dma(0, 0).start()
        ids_dma(1, 1).start()
        ids_dma(0, 0).wait()
        gather_dma(0).start()
        pe_cp.wait()

        def window(w, cur, nxt):
            ids_dma(w + 1, nxt).wait()
            gather_dma(nxt).start()
            gather_dma(cur).wait()
            ids_dma(w + 2, cur).start()
            tab = tabs[cur]
            for g in range(_NG):
                rows = lax.iota(jnp.int32, 16) + g * 16
                pe_off = (2 * w + g // 4) * _D
                pe_half = (pe_v[pl.ds(pe_off, 16)], pe_v[pl.ds(pe_off + 16, 16)])
                acc = jnp.zeros((16,), jnp.float32)
                for d in range(_D):
                    col = jnp.full((16,), d, jnp.int32)
                    tcol = plsc.load_gather(tab, [rows, col])
                    acc = acc + tcol * pe_half[d // 16][d % 16]
                sc_v[pl.ds(w * _W + g * 16, 16)] = acc

        @pl.loop(0, _NW, step=2)
        def _(w):
            window(w, 0, 1)
            window(w + 1, 1, 0)

        gather_dma(0).wait()
        ids_dma(0, 1).wait()

        pltpu.sync_copy(sc_v, out_hbm.at[pl.ds(pair_base, _PAIRS_PER_TILE)])

    return fused_kernel(emb_table, flat_ids, pe)


def kernel(player_state, item_ids, W1, b1, W2, b2, emb_table, temperature):
    pe = _tc_mlp(player_state, W1, b1, W2, b2, temperature)
    ids_p = jnp.pad(item_ids.astype(jnp.int32), ((0, 0), (0, _KP - _K)))
    scores = _sc_fused_score(emb_table, ids_p.reshape(_BKP), pe.reshape(_B * _D))
    return scores.reshape(_B, _KP)[:, :_K]
